# Initial kernel scaffold; baseline (speedup 1.0000x reference)
import jax
import jax.numpy as jnp
from jax.experimental import pallas as pl


def kernel(x, vertex_idx, hyperedge_idx, W_in, b_in, W_c0, b_c0, W_c1, b_c1, W_out, b_out):
    raise NotImplementedError("write your pallas kernel here")


# speedup vs baseline: 2.2734x; 2.2734x over previous
import functools

import jax
import jax.numpy as jnp
from jax import lax
from jax.experimental import pallas as pl
from jax.experimental.pallas import tpu as pltpu
from jax.experimental.pallas import tpu_sc as plsc

NUM_V = 10000
NUM_E = 5000
NNZ = 320000
D_IN = 128
D_HID = 128
D_OUT = 64

NC = 2
NS = 16
NW = NC * NS
CHUNK = 128
CPW = 80
NCHUNKS = NW * CPW
NNZ_PAD = NCHUNKS * CHUNK

V_PAD = 10112
E_PAD = 5120
W = 48
NP = 3


def _make_segsum(src_rows: int, dst_pad: int):
    rpt = dst_pad // NS
    mesh = plsc.VectorSubcoreMesh(core_axis_name="c", subcore_axis_name="s")

    @functools.partial(
        pl.kernel, mesh=mesh,
        compiler_params=pltpu.CompilerParams(use_tc_tiling_on_sc=False),
        out_type=jax.ShapeDtypeStruct((NC, NP, dst_pad, W), jnp.float32),
        scratch_types=[
            pltpu.VMEM((CPW, CHUNK), jnp.int32),
            pltpu.VMEM((CPW, CHUNK), jnp.int32),
            pltpu.VMEM((CHUNK, W), jnp.float32),
            pltpu.VMEM((CHUNK, W), jnp.float32),
            pltpu.VMEM_SHARED((dst_pad, W), jnp.float32),
            pltpu.SemaphoreType.DMA,
            pltpu.SemaphoreType.DMA,
        ])
    def k(t0_hbm, t1_hbm, t2_hbm, gidx_hbm, sidx_hbm, zeros_hbm, out_hbm,
          gidx_v, sidx_v, rows0, rows1, agg_sh, sem0, sem1):
        cid = lax.axis_index("c")
        sid = lax.axis_index("s")
        wid = sid * NC + cid
        r0 = sid * rpt

        pltpu.sync_copy(gidx_hbm.at[pl.ds(wid * CPW, CPW)], gidx_v)
        pltpu.sync_copy(sidx_hbm.at[pl.ds(wid * CPW, CPW)], sidx_v)

        for p, table_hbm in enumerate((t0_hbm, t1_hbm, t2_hbm)):
            pltpu.sync_copy(zeros_hbm.at[pl.ds(r0, rpt)],
                            agg_sh.at[pl.ds(r0, rpt)])
            plsc.subcore_barrier()

            @pl.loop(0, CPW, step=2)
            def _(j):
                c0 = pltpu.async_copy(table_hbm.at[gidx_v.at[j]], rows0, sem0)
                c1 = pltpu.async_copy(table_hbm.at[gidx_v.at[j + 1]], rows1,
                                      sem1)
                c0.wait()
                pltpu.sync_copy(rows0, agg_sh.at[sidx_v.at[j]], add=True)
                c1.wait()
                pltpu.sync_copy(rows1, agg_sh.at[sidx_v.at[j + 1]], add=True)

            plsc.subcore_barrier()
            pltpu.sync_copy(agg_sh.at[pl.ds(r0, rpt)],
                            out_hbm.at[cid].at[p].at[pl.ds(r0, rpt)])
            plsc.subcore_barrier()

    return k


_v2e = _make_segsum(V_PAD, E_PAD)
_e2v = _make_segsum(E_PAD, V_PAD)


def _split_cols(f):
    def body(*refs):
        (o0, o1, o2) = refs[-3:]
        h = f(*refs[:-3])
        ones = jnp.ones((h.shape[0], 16), jnp.float32)
        o0[...] = h[:, 0:W]
        o1[...] = h[:, W:2 * W]
        o2[...] = jnp.concatenate([h[:, 2 * W:D_HID], ones], axis=1)

    return body


_TC_PARAMS = pltpu.CompilerParams(vmem_limit_bytes=100 * 1024 * 1024)


def _tables(body, rows, args):
    shp = [jax.ShapeDtypeStruct((rows, W), jnp.float32)] * 3
    return pl.pallas_call(
        body, out_shape=tuple(shp), compiler_params=_TC_PARAMS)(*args)


def _lin_tables(x, wt, b):
    def f(x_ref, w_ref, b_ref):
        return (jnp.dot(x_ref[...], w_ref[...],
                        preferred_element_type=jnp.float32) + b_ref[...])

    return _tables(_split_cols(f), x.shape[0], (x, wt, b.reshape(1, -1)))


def _combine(a_ref):
    s = a_ref[0] + a_ref[1]
    feat = jnp.concatenate([s[0], s[1], s[2, :, 0:D_HID - 2 * W]], axis=1)
    cnt = s[2, :, D_HID - 2 * W:D_HID - 2 * W + 1]
    return feat, cnt


def _combine_matmul_tables(agg, wt, b):
    def f(a_ref, w_ref, b_ref):
        feat, cnt = _combine(a_ref)
        r = 1.0 / jnp.maximum(cnt, 1.0)
        return (jnp.dot(feat * r, w_ref[...],
                        preferred_element_type=jnp.float32) + b_ref[...])

    return _tables(_split_cols(f), agg.shape[2], (agg, wt, b.reshape(1, -1)))


def _combine_leaky_tables(agg):
    def f(a_ref):
        feat, cnt = _combine(a_ref)
        v = feat * (1.0 / jnp.maximum(cnt, 1.0))
        return jnp.where(v >= 0.0, v, 0.01 * v)

    return _tables(_split_cols(f), agg.shape[2], (agg,))


def _combine_leaky_out(agg, wt, b, out_rows):
    def body(a_ref, w_ref, b_ref, o_ref):
        feat, cnt = _combine(a_ref)
        v = feat * (1.0 / jnp.maximum(cnt, 1.0))
        h = jnp.where(v >= 0.0, v, 0.01 * v)
        o_ref[...] = (jnp.dot(h[0:out_rows], w_ref[...],
                              preferred_element_type=jnp.float32) + b_ref[...])

    return pl.pallas_call(
        body,
        out_shape=jax.ShapeDtypeStruct((out_rows, wt.shape[1]), jnp.float32),
        compiler_params=_TC_PARAMS,
    )(agg, wt, b.reshape(1, -1))


def kernel(x, vertex_idx, hyperedge_idx, W_in, b_in, W_c0, b_c0, W_c1, b_c1,
           W_out, b_out):
    pad = NNZ_PAD - NNZ
    vidx_g = jnp.concatenate(
        [vertex_idx, jnp.zeros((pad,), jnp.int32)]).reshape(NCHUNKS, CHUNK)
    vidx_s = jnp.concatenate(
        [vertex_idx, jnp.full((pad,), V_PAD - 1, jnp.int32)]).reshape(NCHUNKS, CHUNK)
    eidx_g = jnp.concatenate(
        [hyperedge_idx, jnp.zeros((pad,), jnp.int32)]).reshape(NCHUNKS, CHUNK)
    eidx_s = jnp.concatenate(
        [hyperedge_idx, jnp.full((pad,), E_PAD - 1, jnp.int32)]).reshape(NCHUNKS, CHUNK)
    zeros = jnp.zeros((V_PAD, W), jnp.float32)
    x_p = jnp.concatenate([x, jnp.zeros((V_PAD - NUM_V, D_IN), jnp.float32)])

    h0, h1, h2 = _lin_tables(x_p, W_in.T, b_in)
    agg_e = _v2e(h0, h1, h2, vidx_g, eidx_s, zeros)
    e0, e1t, e2t = _combine_matmul_tables(agg_e, W_c0.T, b_c0)
    agg_v = _e2v(e0, e1t, e2t, eidx_g, vidx_s, zeros)
    g0, g1, g2 = _combine_leaky_tables(agg_v)
    agg_e2 = _v2e(g0, g1, g2, vidx_g, eidx_s, zeros)
    f0, f1, f2 = _combine_matmul_tables(agg_e2, W_c1.T, b_c1)
    agg_v2 = _e2v(f0, f1, f2, eidx_g, vidx_s, zeros)
    return _combine_leaky_out(agg_v2, W_out.T, b_out, NUM_V)


# speedup vs baseline: 2.6314x; 1.1575x over previous
import functools

import jax
import jax.numpy as jnp
from jax import lax
from jax.experimental import pallas as pl
from jax.experimental.pallas import tpu as pltpu
from jax.experimental.pallas import tpu_sc as plsc

NUM_V = 10000
NUM_E = 5000
NNZ = 320000
D_IN = 128
D_HID = 128
D_OUT = 64

NC = 2
NS = 16
NW = NC * NS
CHUNK = 128
CPW = 80
NCHUNKS = NW * CPW
NNZ_PAD = NCHUNKS * CHUNK

V_PAD = 10112
E_PAD = 5120
W = 48
NP = 3
NB = 8
GA = 4


def _make_segsum(src_rows: int, dst_pad: int):
    rpt = dst_pad // NS
    mesh = plsc.VectorSubcoreMesh(core_axis_name="c", subcore_axis_name="s")

    @functools.partial(
        pl.kernel, mesh=mesh,
        compiler_params=pltpu.CompilerParams(use_tc_tiling_on_sc=False),
        out_type=jax.ShapeDtypeStruct((NC, NP, dst_pad, W), jnp.float32),
        scratch_types=(
            [pltpu.VMEM((CPW, CHUNK), jnp.int32),
             pltpu.VMEM((CPW, CHUNK), jnp.int32)]
            + [pltpu.VMEM((CHUNK, W), jnp.float32)] * NB
            + [pltpu.VMEM_SHARED((dst_pad, W), jnp.float32)]
            + [pltpu.SemaphoreType.DMA] * (2 * NB)))
    def k(t0_hbm, t1_hbm, t2_hbm, gidx_hbm, sidx_hbm, zeros_hbm, out_hbm,
          gidx_v, sidx_v, *rest):
        bufs = rest[0:NB]
        agg_sh = rest[NB]
        gsem = rest[NB + 1:NB + 1 + NB]
        ssem = rest[NB + 1 + NB:NB + 1 + 2 * NB]
        cid = lax.axis_index("c")
        sid = lax.axis_index("s")
        wid = sid * NC + cid
        r0 = sid * rpt

        pltpu.sync_copy(gidx_hbm.at[pl.ds(wid * CPW, CPW)], gidx_v)
        pltpu.sync_copy(sidx_hbm.at[pl.ds(wid * CPW, CPW)], sidx_v)

        def start_gather(table_hbm, t, bi):
            pltpu.async_copy(table_hbm.at[gidx_v.at[t]], bufs[bi], gsem[bi])

        def wait_gather(table_hbm, t, bi):
            pltpu.make_async_copy(table_hbm.at[gidx_v.at[t]], bufs[bi],
                                  gsem[bi]).wait()

        def start_scatter(t, bi):
            pltpu.async_copy(bufs[bi], agg_sh.at[sidx_v.at[t]],
                             ssem[bi], add=True)

        def wait_scatter(t, bi):
            pltpu.make_async_copy(bufs[bi], agg_sh.at[sidx_v.at[t]],
                                  ssem[bi]).wait()

        for p, table_hbm in enumerate((t0_hbm, t1_hbm, t2_hbm)):
            pltpu.sync_copy(zeros_hbm.at[pl.ds(r0, rpt)],
                            agg_sh.at[pl.ds(r0, rpt)])
            plsc.subcore_barrier()

            for t in range(0, GA):
                start_gather(table_hbm, t, t % NB)
            for t in range(0, GA):
                start_gather(table_hbm, t + GA, (t + GA) % NB)
                wait_gather(table_hbm, t, t % NB)
                start_scatter(t, t % NB)

            @pl.loop(GA, CPW - GA, step=NB)
            def _(j):
                for b in range(NB):
                    t = j + b
                    wait_scatter(t - GA, b)
                    start_gather(table_hbm, t + GA, b)
                    wait_gather(table_hbm, t, (GA + b) % NB)
                    start_scatter(t, (GA + b) % NB)

            for t in range(CPW - GA, CPW):
                wait_scatter(t - GA, (t - GA) % NB)
                wait_gather(table_hbm, t, t % NB)
                start_scatter(t, t % NB)
            for t in range(CPW - GA, CPW):
                wait_scatter(t, t % NB)
            plsc.subcore_barrier()
            pltpu.sync_copy(agg_sh.at[pl.ds(r0, rpt)],
                            out_hbm.at[cid].at[p].at[pl.ds(r0, rpt)])
            plsc.subcore_barrier()

    return k


_v2e = _make_segsum(V_PAD, E_PAD)
_e2v = _make_segsum(E_PAD, V_PAD)


def _split_cols(f):
    def body(*refs):
        (o0, o1, o2) = refs[-3:]
        h = f(*refs[:-3])
        ones = jnp.ones((h.shape[0], 16), jnp.float32)
        o0[...] = h[:, 0:W]
        o1[...] = h[:, W:2 * W]
        o2[...] = jnp.concatenate([h[:, 2 * W:D_HID], ones], axis=1)

    return body


_TC_PARAMS = pltpu.CompilerParams(vmem_limit_bytes=100 * 1024 * 1024)


def _tables(body, rows, args):
    shp = [jax.ShapeDtypeStruct((rows, W), jnp.float32)] * 3
    return pl.pallas_call(
        body, out_shape=tuple(shp), compiler_params=_TC_PARAMS)(*args)


def _lin_tables(x, wt, b):
    def f(x_ref, w_ref, b_ref):
        return (jnp.dot(x_ref[...], w_ref[...],
                        preferred_element_type=jnp.float32) + b_ref[...])

    return _tables(_split_cols(f), x.shape[0], (x, wt, b.reshape(1, -1)))


def _combine(a_ref):
    s = a_ref[0] + a_ref[1]
    feat = jnp.concatenate([s[0], s[1], s[2, :, 0:D_HID - 2 * W]], axis=1)
    cnt = s[2, :, D_HID - 2 * W:D_HID - 2 * W + 1]
    return feat, cnt


def _combine_matmul_tables(agg, wt, b):
    def f(a_ref, w_ref, b_ref):
        feat, cnt = _combine(a_ref)
        r = 1.0 / jnp.maximum(cnt, 1.0)
        return (jnp.dot(feat * r, w_ref[...],
                        preferred_element_type=jnp.float32) + b_ref[...])

    return _tables(_split_cols(f), agg.shape[2], (agg, wt, b.reshape(1, -1)))


def _combine_leaky_tables(agg):
    def f(a_ref):
        feat, cnt = _combine(a_ref)
        v = feat * (1.0 / jnp.maximum(cnt, 1.0))
        return jnp.where(v >= 0.0, v, 0.01 * v)

    return _tables(_split_cols(f), agg.shape[2], (agg,))


def _combine_leaky_out(agg, wt, b, out_rows):
    def body(a_ref, w_ref, b_ref, o_ref):
        feat, cnt = _combine(a_ref)
        v = feat * (1.0 / jnp.maximum(cnt, 1.0))
        h = jnp.where(v >= 0.0, v, 0.01 * v)
        o_ref[...] = (jnp.dot(h[0:out_rows], w_ref[...],
                              preferred_element_type=jnp.float32) + b_ref[...])

    return pl.pallas_call(
        body,
        out_shape=jax.ShapeDtypeStruct((out_rows, wt.shape[1]), jnp.float32),
        compiler_params=_TC_PARAMS,
    )(agg, wt, b.reshape(1, -1))


def kernel(x, vertex_idx, hyperedge_idx, W_in, b_in, W_c0, b_c0, W_c1, b_c1,
           W_out, b_out):
    pad = NNZ_PAD - NNZ
    vidx_g = jnp.concatenate(
        [vertex_idx, jnp.zeros((pad,), jnp.int32)]).reshape(NCHUNKS, CHUNK)
    vidx_s = jnp.concatenate(
        [vertex_idx, jnp.full((pad,), V_PAD - 1, jnp.int32)]).reshape(NCHUNKS, CHUNK)
    eidx_g = jnp.concatenate(
        [hyperedge_idx, jnp.zeros((pad,), jnp.int32)]).reshape(NCHUNKS, CHUNK)
    eidx_s = jnp.concatenate(
        [hyperedge_idx, jnp.full((pad,), E_PAD - 1, jnp.int32)]).reshape(NCHUNKS, CHUNK)
    zeros = jnp.zeros((V_PAD, W), jnp.float32)
    x_p = jnp.concatenate([x, jnp.zeros((V_PAD - NUM_V, D_IN), jnp.float32)])

    h0, h1, h2 = _lin_tables(x_p, W_in.T, b_in)
    agg_e = _v2e(h0, h1, h2, vidx_g, eidx_s, zeros)
    e0, e1t, e2t = _combine_matmul_tables(agg_e, W_c0.T, b_c0)
    agg_v = _e2v(e0, e1t, e2t, eidx_g, vidx_s, zeros)
    g0, g1, g2 = _combine_leaky_tables(agg_v)
    agg_e2 = _v2e(g0, g1, g2, vidx_g, eidx_s, zeros)
    f0, f1, f2 = _combine_matmul_tables(agg_e2, W_c1.T, b_c1)
    agg_v2 = _e2v(f0, f1, f2, eidx_g, vidx_s, zeros)
    return _combine_leaky_out(agg_v2, W_out.T, b_out, NUM_V)


# speedup vs baseline: 2.6355x; 1.0016x over previous
import functools

import jax
import jax.numpy as jnp
from jax import lax
from jax.experimental import pallas as pl
from jax.experimental.pallas import tpu as pltpu
from jax.experimental.pallas import tpu_sc as plsc

NUM_V = 10000
NUM_E = 5000
NNZ = 320000
D_IN = 128
D_HID = 128
D_OUT = 64

NC = 2
NS = 16
NW = NC * NS
CHUNK = 128
CPW = 80
NCHUNKS = NW * CPW
NNZ_PAD = NCHUNKS * CHUNK

V_PAD = 10112
E_PAD = 5120
W = 48
NP = 3
NB = 10
GA = 5


def _make_segsum(src_rows: int, dst_pad: int):
    rpt = dst_pad // NS
    mesh = plsc.VectorSubcoreMesh(core_axis_name="c", subcore_axis_name="s")

    @functools.partial(
        pl.kernel, mesh=mesh,
        compiler_params=pltpu.CompilerParams(use_tc_tiling_on_sc=False),
        out_type=jax.ShapeDtypeStruct((NC, NP, dst_pad, W), jnp.float32),
        scratch_types=(
            [pltpu.VMEM((CPW, CHUNK), jnp.int32),
             pltpu.VMEM((CPW, CHUNK), jnp.int32)]
            + [pltpu.VMEM((CHUNK, W), jnp.float32)] * NB
            + [pltpu.VMEM_SHARED((dst_pad, W), jnp.float32)]
            + [pltpu.SemaphoreType.DMA] * (2 * NB)))
    def k(t0_hbm, t1_hbm, t2_hbm, gidx_hbm, sidx_hbm, zeros_hbm, out_hbm,
          gidx_v, sidx_v, *rest):
        bufs = rest[0:NB]
        agg_sh = rest[NB]
        gsem = rest[NB + 1:NB + 1 + NB]
        ssem = rest[NB + 1 + NB:NB + 1 + 2 * NB]
        cid = lax.axis_index("c")
        sid = lax.axis_index("s")
        wid = sid * NC + cid
        r0 = sid * rpt

        pltpu.sync_copy(gidx_hbm.at[pl.ds(wid * CPW, CPW)], gidx_v)
        pltpu.sync_copy(sidx_hbm.at[pl.ds(wid * CPW, CPW)], sidx_v)

        def start_gather(table_hbm, t, bi):
            pltpu.async_copy(table_hbm.at[gidx_v.at[t]], bufs[bi], gsem[bi])

        def wait_gather(table_hbm, t, bi):
            pltpu.make_async_copy(table_hbm.at[gidx_v.at[t]], bufs[bi],
                                  gsem[bi]).wait()

        def start_scatter(t, bi):
            pltpu.async_copy(bufs[bi], agg_sh.at[sidx_v.at[t]],
                             ssem[bi], add=True)

        def wait_scatter(t, bi):
            pltpu.make_async_copy(bufs[bi], agg_sh.at[sidx_v.at[t]],
                                  ssem[bi]).wait()

        pltpu.sync_copy(zeros_hbm.at[pl.ds(r0, rpt)],
                        agg_sh.at[pl.ds(r0, rpt)])

        tables = (t0_hbm, t1_hbm, t2_hbm)
        for t in range(0, GA):
            start_gather(tables[0], t, t % NB)
        plsc.subcore_barrier()

        for p, table_hbm in enumerate(tables):
            for t in range(0, GA):
                start_gather(table_hbm, t + GA, (t + GA) % NB)
                wait_gather(table_hbm, t, t % NB)
                start_scatter(t, t % NB)

            @pl.loop(GA, CPW - GA, step=NB)
            def _(j, table_hbm=table_hbm):
                for b in range(NB):
                    t = j + b
                    wait_scatter(t - GA, b)
                    start_gather(table_hbm, t + GA, b)
                    wait_gather(table_hbm, t, (GA + b) % NB)
                    start_scatter(t, (GA + b) % NB)

            for t in range(CPW - GA, CPW):
                wait_scatter(t - GA, (t - GA) % NB)
                wait_gather(table_hbm, t, t % NB)
                start_scatter(t, t % NB)
            for t in range(CPW - GA, CPW):
                wait_scatter(t, t % NB)
            if p + 1 < NP:
                for t in range(0, GA):
                    start_gather(tables[p + 1], t, t % NB)
            plsc.subcore_barrier()
            pltpu.sync_copy(agg_sh.at[pl.ds(r0, rpt)],
                            out_hbm.at[cid].at[p].at[pl.ds(r0, rpt)])
            if p + 1 < NP:
                pltpu.sync_copy(zeros_hbm.at[pl.ds(r0, rpt)],
                                agg_sh.at[pl.ds(r0, rpt)])
                plsc.subcore_barrier()

    return k


_v2e = _make_segsum(V_PAD, E_PAD)
_e2v = _make_segsum(E_PAD, V_PAD)


def _split_cols(f):
    def body(*refs):
        (o0, o1, o2) = refs[-3:]
        h = f(*refs[:-3])
        ones = jnp.ones((h.shape[0], 16), jnp.float32)
        o0[...] = h[:, 0:W]
        o1[...] = h[:, W:2 * W]
        o2[...] = jnp.concatenate([h[:, 2 * W:D_HID], ones], axis=1)

    return body


_TC_PARAMS = pltpu.CompilerParams(vmem_limit_bytes=100 * 1024 * 1024)


def _tables(body, rows, args):
    shp = [jax.ShapeDtypeStruct((rows, W), jnp.float32)] * 3
    return pl.pallas_call(
        body, out_shape=tuple(shp), compiler_params=_TC_PARAMS)(*args)


def _lin_tables(x, wt, b):
    def f(x_ref, w_ref, b_ref):
        return (jnp.dot(x_ref[...], w_ref[...],
                        preferred_element_type=jnp.float32) + b_ref[...])

    return _tables(_split_cols(f), x.shape[0], (x, wt, b.reshape(1, -1)))


def _combine(a_ref):
    s = a_ref[0] + a_ref[1]
    feat = jnp.concatenate([s[0], s[1], s[2, :, 0:D_HID - 2 * W]], axis=1)
    cnt = s[2, :, D_HID - 2 * W:D_HID - 2 * W + 1]
    return feat, cnt


def _combine_matmul_tables(agg, wt, b):
    def f(a_ref, w_ref, b_ref):
        feat, cnt = _combine(a_ref)
        r = 1.0 / jnp.maximum(cnt, 1.0)
        return (jnp.dot(feat * r, w_ref[...],
                        preferred_element_type=jnp.float32) + b_ref[...])

    return _tables(_split_cols(f), agg.shape[2], (agg, wt, b.reshape(1, -1)))


def _combine_leaky_tables(agg):
    def f(a_ref):
        feat, cnt = _combine(a_ref)
        v = feat * (1.0 / jnp.maximum(cnt, 1.0))
        return jnp.where(v >= 0.0, v, 0.01 * v)

    return _tables(_split_cols(f), agg.shape[2], (agg,))


def _combine_leaky_out(agg, wt, b, out_rows):
    def body(a_ref, w_ref, b_ref, o_ref):
        feat, cnt = _combine(a_ref)
        v = feat * (1.0 / jnp.maximum(cnt, 1.0))
        h = jnp.where(v >= 0.0, v, 0.01 * v)
        o_ref[...] = (jnp.dot(h[0:out_rows], w_ref[...],
                              preferred_element_type=jnp.float32) + b_ref[...])

    return pl.pallas_call(
        body,
        out_shape=jax.ShapeDtypeStruct((out_rows, wt.shape[1]), jnp.float32),
        compiler_params=_TC_PARAMS,
    )(agg, wt, b.reshape(1, -1))


def kernel(x, vertex_idx, hyperedge_idx, W_in, b_in, W_c0, b_c0, W_c1, b_c1,
           W_out, b_out):
    pad = NNZ_PAD - NNZ
    vidx_g = jnp.concatenate(
        [vertex_idx, jnp.zeros((pad,), jnp.int32)]).reshape(NCHUNKS, CHUNK)
    vidx_s = jnp.concatenate(
        [vertex_idx, jnp.full((pad,), V_PAD - 1, jnp.int32)]).reshape(NCHUNKS, CHUNK)
    eidx_g = jnp.concatenate(
        [hyperedge_idx, jnp.zeros((pad,), jnp.int32)]).reshape(NCHUNKS, CHUNK)
    eidx_s = jnp.concatenate(
        [hyperedge_idx, jnp.full((pad,), E_PAD - 1, jnp.int32)]).reshape(NCHUNKS, CHUNK)
    zeros = jnp.zeros((V_PAD, W), jnp.float32)
    x_p = jnp.concatenate([x, jnp.zeros((V_PAD - NUM_V, D_IN), jnp.float32)])

    h0, h1, h2 = _lin_tables(x_p, W_in.T, b_in)
    agg_e = _v2e(h0, h1, h2, vidx_g, eidx_s, zeros)
    e0, e1t, e2t = _combine_matmul_tables(agg_e, W_c0.T, b_c0)
    agg_v = _e2v(e0, e1t, e2t, eidx_g, vidx_s, zeros)
    g0, g1, g2 = _combine_leaky_tables(agg_v)
    agg_e2 = _v2e(g0, g1, g2, vidx_g, eidx_s, zeros)
    f0, f1, f2 = _combine_matmul_tables(agg_e2, W_c1.T, b_c1)
    agg_v2 = _e2v(f0, f1, f2, eidx_g, vidx_s, zeros)
    return _combine_leaky_out(agg_v2, W_out.T, b_out, NUM_V)


# speedup vs baseline: 9.1630x; 3.4767x over previous
import functools

import jax
import jax.numpy as jnp
from jax import lax
from jax.experimental import pallas as pl
from jax.experimental.pallas import tpu as pltpu
from jax.experimental.pallas import tpu_sc as plsc

NUM_V = 10000
NUM_E = 5000
NNZ = 320000
D_IN = 128
D_HID = 128
D_OUT = 64

NC = 2
NS = 16
NW = NC * NS
CHUNK = 128
CPW = 80
NCHUNKS = NW * CPW
NNZ_PAD = NCHUNKS * CHUNK

V_PAD = 10112
E_PAD = 5120
W = 48
NP = 3
NB = 10
GA = 5


def _make_segsum(src_rows: int, dst_pad: int):
    rpt = dst_pad // NS
    mesh = plsc.VectorSubcoreMesh(core_axis_name="c", subcore_axis_name="s")

    @functools.partial(
        pl.kernel, mesh=mesh,
        compiler_params=pltpu.CompilerParams(use_tc_tiling_on_sc=False),
        out_type=jax.ShapeDtypeStruct((NC, NP, dst_pad, W), jnp.float32),
        scratch_types=(
            [pltpu.VMEM((CPW, CHUNK), jnp.int32),
             pltpu.VMEM((CPW, CHUNK), jnp.int32)]
            + [pltpu.VMEM((CHUNK, W), jnp.float32)] * NB
            + [pltpu.VMEM_SHARED((dst_pad, W), jnp.float32)]
            + [pltpu.SemaphoreType.DMA] * (2 * NB)))
    def k(t0_hbm, t1_hbm, t2_hbm, gidx_hbm, sidx_hbm, zeros_hbm, out_hbm,
          gidx_v, sidx_v, *rest):
        bufs = rest[0:NB]
        agg_sh = rest[NB]
        gsem = rest[NB + 1:NB + 1 + NB]
        ssem = rest[NB + 1 + NB:NB + 1 + 2 * NB]
        cid = lax.axis_index("c")
        sid = lax.axis_index("s")
        wid = sid * NC + cid
        r0 = sid * rpt

        pltpu.sync_copy(gidx_hbm.at[pl.ds(wid * CPW, CPW)], gidx_v)
        pltpu.sync_copy(sidx_hbm.at[pl.ds(wid * CPW, CPW)], sidx_v)

        def start_gather(table_hbm, t, bi):
            pltpu.async_copy(table_hbm.at[gidx_v.at[t]], bufs[bi], gsem[bi])

        def wait_gather(table_hbm, t, bi):
            pltpu.make_async_copy(table_hbm.at[gidx_v.at[t]], bufs[bi],
                                  gsem[bi]).wait()

        def start_scatter(t, bi):
            pltpu.async_copy(bufs[bi], agg_sh.at[sidx_v.at[t]],
                             ssem[bi], add=True)

        def wait_scatter(t, bi):
            pltpu.make_async_copy(bufs[bi], agg_sh.at[sidx_v.at[t]],
                                  ssem[bi]).wait()

        pltpu.sync_copy(zeros_hbm.at[pl.ds(r0, rpt)],
                        agg_sh.at[pl.ds(r0, rpt)])

        tables = (t0_hbm, t1_hbm, t2_hbm)
        for t in range(0, GA):
            start_gather(tables[0], t, t % NB)
        plsc.subcore_barrier()

        for p, table_hbm in enumerate(tables):
            for t in range(0, GA):
                start_gather(table_hbm, t + GA, (t + GA) % NB)
                wait_gather(table_hbm, t, t % NB)
                start_scatter(t, t % NB)

            @pl.loop(GA, CPW - GA, step=NB)
            def _(j, table_hbm=table_hbm):
                for b in range(NB):
                    t = j + b
                    wait_scatter(t - GA, b)
                    start_gather(table_hbm, t + GA, b)
                    wait_gather(table_hbm, t, (GA + b) % NB)
                    start_scatter(t, (GA + b) % NB)

            for t in range(CPW - GA, CPW):
                wait_scatter(t - GA, (t - GA) % NB)
                wait_gather(table_hbm, t, t % NB)
                start_scatter(t, t % NB)
            for t in range(CPW - GA, CPW):
                wait_scatter(t, t % NB)
            if p + 1 < NP:
                for t in range(0, GA):
                    start_gather(tables[p + 1], t, t % NB)
            plsc.subcore_barrier()
            pltpu.sync_copy(agg_sh.at[pl.ds(r0, rpt)],
                            out_hbm.at[cid].at[p].at[pl.ds(r0, rpt)])
            if p + 1 < NP:
                pltpu.sync_copy(zeros_hbm.at[pl.ds(r0, rpt)],
                                agg_sh.at[pl.ds(r0, rpt)])
                plsc.subcore_barrier()

    return k


_v2e = _make_segsum(V_PAD, E_PAD)
_e2v = _make_segsum(E_PAD, V_PAD)


def _split_cols(f):
    def body(*refs):
        (o0, o1, o2) = refs[-3:]
        h = f(*refs[:-3])
        ones = jnp.ones((h.shape[0], 16), jnp.float32)
        o0[...] = h[:, 0:W]
        o1[...] = h[:, W:2 * W]
        o2[...] = jnp.concatenate([h[:, 2 * W:D_HID], ones], axis=1)

    return body


_TC_PARAMS = pltpu.CompilerParams(vmem_limit_bytes=100 * 1024 * 1024)


def _tables(body, rows, args):
    shp = [jax.ShapeDtypeStruct((rows, W), jnp.float32)] * 3
    return pl.pallas_call(
        body, out_shape=tuple(shp), compiler_params=_TC_PARAMS)(*args)


def _lin_tables(x, wt, b):
    def f(x_ref, w_ref, b_ref):
        return (jnp.dot(x_ref[...], w_ref[...],
                        preferred_element_type=jnp.float32) + b_ref[...])

    return _tables(_split_cols(f), x.shape[0], (x, wt, b.reshape(1, -1)))


def _combine(a_ref):
    s = a_ref[0] + a_ref[1]
    feat = jnp.concatenate([s[0], s[1], s[2, :, 0:D_HID - 2 * W]], axis=1)
    cnt = s[2, :, D_HID - 2 * W:D_HID - 2 * W + 1]
    return feat, cnt


def _combine_matmul_tables(agg, wt, b):
    def f(a_ref, w_ref, b_ref):
        feat, cnt = _combine(a_ref)
        r = 1.0 / jnp.maximum(cnt, 1.0)
        return (jnp.dot(feat * r, w_ref[...],
                        preferred_element_type=jnp.float32) + b_ref[...])

    return _tables(_split_cols(f), agg.shape[2], (agg, wt, b.reshape(1, -1)))


def _combine_leaky_tables(agg):
    def f(a_ref):
        feat, cnt = _combine(a_ref)
        v = feat * (1.0 / jnp.maximum(cnt, 1.0))
        return jnp.where(v >= 0.0, v, 0.01 * v)

    return _tables(_split_cols(f), agg.shape[2], (agg,))


def _combine_leaky_out(agg, wt, b, out_rows):
    def body(a_ref, w_ref, b_ref, o_ref):
        feat, cnt = _combine(a_ref)
        v = feat * (1.0 / jnp.maximum(cnt, 1.0))
        h = jnp.where(v >= 0.0, v, 0.01 * v)
        o_ref[...] = (jnp.dot(h[0:out_rows], w_ref[...],
                              preferred_element_type=jnp.float32) + b_ref[...])

    return pl.pallas_call(
        body,
        out_shape=jax.ShapeDtypeStruct((out_rows, wt.shape[1]), jnp.float32),
        compiler_params=_TC_PARAMS,
    )(agg, wt, b.reshape(1, -1))


def kernel(x, vertex_idx, hyperedge_idx, W_in, b_in, W_c0, b_c0, W_c1, b_c1,
           W_out, b_out):
    pad = NNZ_PAD - NNZ
    ar = jnp.arange(pad, dtype=jnp.int32)
    vidx_g = jnp.concatenate(
        [vertex_idx, ar % NUM_V]).reshape(NCHUNKS, CHUNK)
    vidx_s = jnp.concatenate(
        [vertex_idx, NUM_V + ar % (V_PAD - NUM_V)]).reshape(NCHUNKS, CHUNK)
    eidx_g = jnp.concatenate(
        [hyperedge_idx, ar % NUM_E]).reshape(NCHUNKS, CHUNK)
    eidx_s = jnp.concatenate(
        [hyperedge_idx, NUM_E + ar % (E_PAD - NUM_E)]).reshape(NCHUNKS, CHUNK)
    zeros = jnp.zeros((V_PAD, W), jnp.float32)
    x_p = jnp.concatenate([x, jnp.zeros((V_PAD - NUM_V, D_IN), jnp.float32)])

    h0, h1, h2 = _lin_tables(x_p, W_in.T, b_in)
    agg_e = _v2e(h0, h1, h2, vidx_g, eidx_s, zeros)
    e0, e1t, e2t = _combine_matmul_tables(agg_e, W_c0.T, b_c0)
    agg_v = _e2v(e0, e1t, e2t, eidx_g, vidx_s, zeros)
    g0, g1, g2 = _combine_leaky_tables(agg_v)
    agg_e2 = _v2e(g0, g1, g2, vidx_g, eidx_s, zeros)
    f0, f1, f2 = _combine_matmul_tables(agg_e2, W_c1.T, b_c1)
    agg_v2 = _e2v(f0, f1, f2, eidx_g, vidx_s, zeros)
    return _combine_leaky_out(agg_v2, W_out.T, b_out, NUM_V)


# speedup vs baseline: 10.1546x; 1.1082x over previous
import functools

import jax
import jax.numpy as jnp
from jax import lax
from jax.experimental import pallas as pl
from jax.experimental.pallas import tpu as pltpu
from jax.experimental.pallas import tpu_sc as plsc

NUM_V = 10000
NUM_E = 5000
NNZ = 320000
D_IN = 128
D_HID = 128
D_OUT = 64

NC = 2
NS = 16
NW = NC * NS
CHUNK = 128
CPW = 80
NCHUNKS = NW * CPW
NNZ_PAD = NCHUNKS * CHUNK

V_PAD = 10112
E_PAD = 5120
W = 48
NP = 3
NB = 10
GA = 5


def _make_segsum(src_rows: int, dst_pad: int):
    rpt = dst_pad // NS
    mesh = plsc.VectorSubcoreMesh(core_axis_name="c", subcore_axis_name="s")

    @functools.partial(
        pl.kernel, mesh=mesh,
        compiler_params=pltpu.CompilerParams(use_tc_tiling_on_sc=False),
        out_type=jax.ShapeDtypeStruct((NC * NP * dst_pad, 128), jnp.float32),
        scratch_types=(
            [pltpu.VMEM((CPW, CHUNK), jnp.int32),
             pltpu.VMEM((CPW, CHUNK), jnp.int32)]
            + [pltpu.VMEM((CHUNK, W), jnp.float32)] * NB
            + [pltpu.VMEM_SHARED((dst_pad, W), jnp.float32)]
            + [pltpu.SemaphoreType.DMA] * (2 * NB)))
    def k(t0_hbm, t1_hbm, t2_hbm, gidx_hbm, sidx_hbm, zeros_hbm, out_hbm,
          gidx_v, sidx_v, *rest):
        bufs = rest[0:NB]
        agg_sh = rest[NB]
        gsem = rest[NB + 1:NB + 1 + NB]
        ssem = rest[NB + 1 + NB:NB + 1 + 2 * NB]
        cid = lax.axis_index("c")
        sid = lax.axis_index("s")
        wid = sid * NC + cid
        r0 = sid * rpt

        pltpu.sync_copy(gidx_hbm.at[pl.ds(wid * CPW, CPW)], gidx_v)
        pltpu.sync_copy(sidx_hbm.at[pl.ds(wid * CPW, CPW)], sidx_v)

        def start_gather(table_hbm, t, bi):
            pltpu.async_copy(table_hbm.at[gidx_v.at[t]], bufs[bi], gsem[bi])

        def wait_gather(table_hbm, t, bi):
            pltpu.make_async_copy(table_hbm.at[gidx_v.at[t]], bufs[bi],
                                  gsem[bi]).wait()

        def start_scatter(t, bi):
            pltpu.async_copy(bufs[bi], agg_sh.at[sidx_v.at[t]],
                             ssem[bi], add=True)

        def wait_scatter(t, bi):
            pltpu.make_async_copy(bufs[bi], agg_sh.at[sidx_v.at[t]],
                                  ssem[bi]).wait()

        pltpu.sync_copy(zeros_hbm.at[pl.ds(r0, rpt)],
                        agg_sh.at[pl.ds(r0, rpt)])

        tables = (t0_hbm, t1_hbm, t2_hbm)
        for t in range(0, GA):
            start_gather(tables[0], t, t % NB)
        plsc.subcore_barrier()

        for p, table_hbm in enumerate(tables):
            for t in range(0, GA):
                start_gather(table_hbm, t + GA, (t + GA) % NB)
                wait_gather(table_hbm, t, t % NB)
                start_scatter(t, t % NB)

            @pl.loop(GA, CPW - GA, step=NB)
            def _(j, table_hbm=table_hbm):
                for b in range(NB):
                    t = j + b
                    wait_scatter(t - GA, b)
                    start_gather(table_hbm, t + GA, b)
                    wait_gather(table_hbm, t, (GA + b) % NB)
                    start_scatter(t, (GA + b) % NB)

            for t in range(CPW - GA, CPW):
                wait_scatter(t - GA, (t - GA) % NB)
                wait_gather(table_hbm, t, t % NB)
                start_scatter(t, t % NB)
            for t in range(CPW - GA, CPW):
                wait_scatter(t, t % NB)
            if p + 1 < NP:
                for t in range(0, GA):
                    start_gather(tables[p + 1], t, t % NB)
            plsc.subcore_barrier()
            pltpu.sync_copy(
                agg_sh.at[pl.ds(r0, rpt)],
                out_hbm.at[pl.ds((cid * NP + p) * dst_pad + r0, rpt),
                           pl.ds(0, W)])
            if p + 1 < NP:
                pltpu.sync_copy(zeros_hbm.at[pl.ds(r0, rpt)],
                                agg_sh.at[pl.ds(r0, rpt)])
                plsc.subcore_barrier()

    return k


_v2e = _make_segsum(V_PAD, E_PAD)
_e2v = _make_segsum(E_PAD, V_PAD)


def _split_cols(f):
    def body(*refs):
        (o0, o1, o2) = refs[-3:]
        h = f(*refs[:-3])
        ones = jnp.ones((h.shape[0], 16), jnp.float32)
        o0[...] = h[:, 0:W]
        o1[...] = h[:, W:2 * W]
        o2[...] = jnp.concatenate([h[:, 2 * W:D_HID], ones], axis=1)

    return body


_TC_PARAMS = pltpu.CompilerParams(vmem_limit_bytes=100 * 1024 * 1024)


def _tables(body, rows, args):
    shp = [jax.ShapeDtypeStruct((rows, W), jnp.float32)] * 3
    return pl.pallas_call(
        body, out_shape=tuple(shp), compiler_params=_TC_PARAMS)(*args)


def _lin_tables(x, wt, b):
    def f(x_ref, w_ref, b_ref):
        return (jnp.dot(x_ref[...], w_ref[...],
                        preferred_element_type=jnp.float32) + b_ref[...])

    return _tables(_split_cols(f), x.shape[0], (x, wt, b.reshape(1, -1)))


def _combine(a_ref):
    rows = a_ref.shape[0] // (NC * NP)

    def blk(c, p):
        return a_ref[(c * NP + p) * rows:(c * NP + p + 1) * rows]

    f0 = blk(0, 0)[:, 0:W] + blk(1, 0)[:, 0:W]
    f1 = blk(0, 1)[:, 0:W] + blk(1, 1)[:, 0:W]
    c = D_HID - 2 * W
    f2 = blk(0, 2)[:, 0:c] + blk(1, 2)[:, 0:c]
    cnt = blk(0, 2)[:, c:c + 1] + blk(1, 2)[:, c:c + 1]
    feat = jnp.concatenate([f0, f1, f2], axis=1)
    return feat, cnt


def _combine_matmul_tables(agg, wt, b):
    def f(a_ref, w_ref, b_ref):
        feat, cnt = _combine(a_ref)
        r = 1.0 / jnp.maximum(cnt, 1.0)
        return (jnp.dot(feat * r, w_ref[...],
                        preferred_element_type=jnp.float32) + b_ref[...])

    rows = agg.shape[0] // (NC * NP)
    return _tables(_split_cols(f), rows, (agg, wt, b.reshape(1, -1)))


def _combine_leaky_tables(agg):
    def f(a_ref):
        feat, cnt = _combine(a_ref)
        v = feat * (1.0 / jnp.maximum(cnt, 1.0))
        return jnp.where(v >= 0.0, v, 0.01 * v)

    rows = agg.shape[0] // (NC * NP)
    return _tables(_split_cols(f), rows, (agg,))


def _combine_leaky_out(agg, wt, b, out_rows):
    def body(a_ref, w_ref, b_ref, o_ref):
        feat, cnt = _combine(a_ref)
        v = feat * (1.0 / jnp.maximum(cnt, 1.0))
        h = jnp.where(v >= 0.0, v, 0.01 * v)
        o_ref[...] = (jnp.dot(h[0:out_rows], w_ref[...],
                              preferred_element_type=jnp.float32) + b_ref[...])

    return pl.pallas_call(
        body,
        out_shape=jax.ShapeDtypeStruct((out_rows, wt.shape[1]), jnp.float32),
        compiler_params=_TC_PARAMS,
    )(agg, wt, b.reshape(1, -1))


def kernel(x, vertex_idx, hyperedge_idx, W_in, b_in, W_c0, b_c0, W_c1, b_c1,
           W_out, b_out):
    pad = NNZ_PAD - NNZ
    ar = jnp.arange(pad, dtype=jnp.int32)
    vidx_g = jnp.concatenate(
        [vertex_idx, ar % NUM_V]).reshape(NCHUNKS, CHUNK)
    vidx_s = jnp.concatenate(
        [vertex_idx, NUM_V + ar % (V_PAD - NUM_V)]).reshape(NCHUNKS, CHUNK)
    eidx_g = jnp.concatenate(
        [hyperedge_idx, ar % NUM_E]).reshape(NCHUNKS, CHUNK)
    eidx_s = jnp.concatenate(
        [hyperedge_idx, NUM_E + ar % (E_PAD - NUM_E)]).reshape(NCHUNKS, CHUNK)
    zeros = jnp.zeros((V_PAD, W), jnp.float32)
    x_p = jnp.concatenate([x, jnp.zeros((V_PAD - NUM_V, D_IN), jnp.float32)])

    h0, h1, h2 = _lin_tables(x_p, W_in.T, b_in)
    agg_e = _v2e(h0, h1, h2, vidx_g, eidx_s, zeros)
    e0, e1t, e2t = _combine_matmul_tables(agg_e, W_c0.T, b_c0)
    agg_v = _e2v(e0, e1t, e2t, eidx_g, vidx_s, zeros)
    g0, g1, g2 = _combine_leaky_tables(agg_v)
    agg_e2 = _v2e(g0, g1, g2, vidx_g, eidx_s, zeros)
    f0, f1, f2 = _combine_matmul_tables(agg_e2, W_c1.T, b_c1)
    agg_v2 = _e2v(f0, f1, f2, eidx_g, vidx_s, zeros)
    return _combine_leaky_out(agg_v2, W_out.T, b_out, NUM_V)
